# baseline (speedup 1.0000x reference)
import functools

import jax
import jax.numpy as jnp
from jax import lax
from jax.experimental import pallas as pl
from jax.experimental.pallas import tpu as pltpu
from jax.experimental.pallas import tpu_sc as plsc

_SIZE = 2.0
_RES = 128
_D = 16
_N = 1048576
_NW = 32
_PER_W = _N // _NW
_CHUNK = 2048
_NCHUNK = _PER_W // _CHUNK
_NB = _CHUNK // 128
_L = 16
_UNROLL = 4
_PLANE_WORDS = 16384 * 8 * 128


def _sc_body(x_hbm, y_hbm, z_hbm, tab_hbm, out_hbm, *scratch):
    (x0, y0, z0, w0_, p00, p01, x1, y1, z1, w1_, p10, p11,
     sa0, sg0, so0, sa1, sg1, so1) = scratch
    bufs = ((x0, y0, z0, w0_, p00, p01), (x1, y1, z1, w1_, p10, p11))
    sems = ((sa0, sg0, so0), (sa1, sg1, so1))

    wid = lax.axis_index("s") * 2 + lax.axis_index("c")
    base = wid * _PER_W

    def start_xyz(ci):
        xv, yv, zv = bufs[ci % 2][:3]
        sem = sems[ci % 2][0]
        pbase = base + ci * _CHUNK
        return [
            pltpu.async_copy(h.at[pl.ds(pbase, _CHUNK)], v, sem)
            for h, v in ((x_hbm, xv), (y_hbm, yv), (z_hbm, zv))
        ]

    def compute(ci):
        xv, yv, zv, wv = bufs[ci % 2][:4]

        def quant(v):
            n = jnp.clip(v * (1.0 / _SIZE) + 0.5, 0.0, 1.0 - 1e-6)
            return (n * _RES).astype(jnp.int32)

        def grp(g, c):
            for u in range(_UNROLL):
                gg = g * _UNROLL + u
                b = gg // 8
                lo = (gg % 8) * _L
                off = gg * _L
                x = quant(xv[pl.ds(off, _L)])
                y = quant(yv[pl.ds(off, _L)])
                z = quant(zv[pl.ds(off, _L)])
                r = (x * _RES + y) * _RES + z
                w = ((r >> 7) << 10) + (r & 127)
                for s in range(8):
                    wv[pl.ds(b * 1024 + s * 128 + lo, _L)] = w + s * 128
            return c

        lax.fori_loop(0, _CHUNK // (_L * _UNROLL), grp, 0)

    def start_gathers(ci):
        wv, pa, pb = bufs[ci % 2][3:6]
        sem = sems[ci % 2][1]
        half = _NB * 512
        return [
            pltpu.async_copy(
                tab_hbm.at[i].at[wv.at[pl.ds(h * half, half)]],
                pv.at[pl.ds(h * half, half)], sem)
            for i, pv in ((0, pa), (1, pb))
            for h in (0, 1)
        ]

    def start_out(ci):
        pa, pb = bufs[ci % 2][4:6]
        sem = sems[ci % 2][2]
        w0 = ((base + ci * _CHUNK) // 128) * 1024
        return [
            pltpu.async_copy(pa, out_hbm.at[0].at[pl.ds(w0, _NB * 1024)], sem),
            pltpu.async_copy(pb, out_hbm.at[1].at[pl.ds(w0, _NB * 1024)], sem),
        ]

    a_descs = {0: start_xyz(0), 1: start_xyz(1)}
    c_descs = {}
    d_descs = {}
    for ci in range(_NCHUNK):
        for d in a_descs.pop(ci):
            d.wait()
        if ci >= 2:
            for d in d_descs.pop(ci - 2):
                d.wait()
        compute(ci)
        c_descs[ci] = start_gathers(ci)
        if ci >= 1:
            for d in c_descs.pop(ci - 1):
                d.wait()
            d_descs[ci - 1] = start_out(ci - 1)
        if ci + 2 < _NCHUNK:
            a_descs[ci + 2] = start_xyz(ci + 2)
    last = _NCHUNK - 1
    for d in c_descs.pop(last):
        d.wait()
    d_descs[last] = start_out(last)
    for d in d_descs.pop(last - 1):
        d.wait()
    for d in d_descs.pop(last):
        d.wait()


@jax.jit
def kernel(xyz, table):
    mesh = plsc.VectorSubcoreMesh(core_axis_name="c", subcore_axis_name="s")
    run = functools.partial(
        pl.kernel,
        name="occ_gather",
        out_type=jax.ShapeDtypeStruct((2, _N * 8), jnp.float32),
        mesh=mesh,
        compiler_params=pltpu.CompilerParams(use_tc_tiling_on_sc=False),
        scratch_types=[
            pltpu.VMEM((_CHUNK,), jnp.float32),
            pltpu.VMEM((_CHUNK,), jnp.float32),
            pltpu.VMEM((_CHUNK,), jnp.float32),
            pltpu.VMEM((_NB * 1024,), jnp.int32),
            pltpu.VMEM((_NB * 1024,), jnp.float32),
            pltpu.VMEM((_NB * 1024,), jnp.float32),
            pltpu.VMEM((_CHUNK,), jnp.float32),
            pltpu.VMEM((_CHUNK,), jnp.float32),
            pltpu.VMEM((_CHUNK,), jnp.float32),
            pltpu.VMEM((_NB * 1024,), jnp.int32),
            pltpu.VMEM((_NB * 1024,), jnp.float32),
            pltpu.VMEM((_NB * 1024,), jnp.float32),
            pltpu.SemaphoreType.DMA,
            pltpu.SemaphoreType.DMA,
            pltpu.SemaphoreType.DMA,
            pltpu.SemaphoreType.DMA,
            pltpu.SemaphoreType.DMA,
            pltpu.SemaphoreType.DMA,
        ],
    )(_sc_body)
    tabflat = (table.reshape(16384, 128, 2, 8)
               .transpose(2, 0, 3, 1)
               .reshape(2, _PLANE_WORDS))
    out2 = run(xyz[:, 0], xyz[:, 1], xyz[:, 2], tabflat)
    out4 = out2.reshape(2, _N // 128, 8, 128)
    return out4.transpose(1, 3, 0, 2).reshape(_N, _D)


# speedup vs baseline: 1.0021x; 1.0021x over previous
import functools

import jax
import jax.numpy as jnp
from jax import lax
from jax.experimental import pallas as pl
from jax.experimental.pallas import tpu as pltpu
from jax.experimental.pallas import tpu_sc as plsc

_SIZE = 2.0
_RES = 128
_D = 16
_N = 1048576
_NW = 32
_PER_W = _N // _NW
_CHUNK = 2048
_NCHUNK = _PER_W // _CHUNK
_NB = _CHUNK // 128
_L = 16
_UNROLL = 4
_PLANE_WORDS = 16384 * 8 * 128


def _sc_body(x_hbm, y_hbm, z_hbm, tab_hbm, out_hbm, *scratch):
    (x0, y0, z0, w0_, p00, p01, x1, y1, z1, w1_, p10, p11,
     sa0, sg0, so0, sa1, sg1, so1) = scratch
    bufs = ((x0, y0, z0, w0_, p00, p01), (x1, y1, z1, w1_, p10, p11))
    sems = ((sa0, sg0, so0), (sa1, sg1, so1))

    wid = lax.axis_index("s") * 2 + lax.axis_index("c")
    base = wid * _PER_W

    def start_xyz(ci):
        xv, yv, zv = bufs[ci % 2][:3]
        sem = sems[ci % 2][0]
        pbase = base + ci * _CHUNK
        return [
            pltpu.async_copy(h.at[pl.ds(pbase, _CHUNK)], v, sem)
            for h, v in ((x_hbm, xv), (y_hbm, yv), (z_hbm, zv))
        ]

    def compute(ci):
        xv, yv, zv, wv = bufs[ci % 2][:4]

        def quant(v):
            n = jnp.clip(v * (1.0 / _SIZE) + 0.5, 0.0, 1.0 - 1e-6)
            return (n * _RES).astype(jnp.int32)

        def grp(g, c):
            for u in range(_UNROLL):
                gg = g * _UNROLL + u
                b = gg // 8
                lo = (gg % 8) * _L
                off = gg * _L
                x = quant(xv[pl.ds(off, _L)])
                y = quant(yv[pl.ds(off, _L)])
                z = quant(zv[pl.ds(off, _L)])
                r = (x * _RES + y) * _RES + z
                w = ((r >> 7) << 10) + (r & 127)
                for s in range(8):
                    wv[pl.ds(b * 1024 + s * 128 + lo, _L)] = w + s * 128
            return c

        lax.fori_loop(0, _CHUNK // (_L * _UNROLL), grp, 0)

    def start_gathers(ci):
        wv, pa, pb = bufs[ci % 2][3:6]
        sem = sems[ci % 2][1]
        return [
            pltpu.async_copy(tab_hbm.at[0].at[wv], pa, sem),
            pltpu.async_copy(tab_hbm.at[1].at[wv], pb, sem),
        ]

    def start_out(ci):
        pa, pb = bufs[ci % 2][4:6]
        sem = sems[ci % 2][2]
        w0 = ((base + ci * _CHUNK) // 128) * 1024
        return [
            pltpu.async_copy(pa, out_hbm.at[0].at[pl.ds(w0, _NB * 1024)], sem),
            pltpu.async_copy(pb, out_hbm.at[1].at[pl.ds(w0, _NB * 1024)], sem),
        ]

    a_descs = {0: start_xyz(0), 1: start_xyz(1)}
    c_descs = {}
    d_descs = {}
    for ci in range(_NCHUNK):
        for d in a_descs.pop(ci):
            d.wait()
        if ci >= 2:
            for d in d_descs.pop(ci - 2):
                d.wait()
        compute(ci)
        c_descs[ci] = start_gathers(ci)
        if ci >= 1:
            for d in c_descs.pop(ci - 1):
                d.wait()
            d_descs[ci - 1] = start_out(ci - 1)
        if ci + 2 < _NCHUNK:
            a_descs[ci + 2] = start_xyz(ci + 2)
    last = _NCHUNK - 1
    for d in c_descs.pop(last):
        d.wait()
    d_descs[last] = start_out(last)
    for d in d_descs.pop(last - 1):
        d.wait()
    for d in d_descs.pop(last):
        d.wait()


@jax.jit
def kernel(xyz, table):
    mesh = plsc.VectorSubcoreMesh(core_axis_name="c", subcore_axis_name="s")
    run = functools.partial(
        pl.kernel,
        name="occ_gather",
        out_type=jax.ShapeDtypeStruct((2, _N * 8), jnp.float32),
        mesh=mesh,
        compiler_params=pltpu.CompilerParams(use_tc_tiling_on_sc=False),
        scratch_types=[
            pltpu.VMEM((_CHUNK,), jnp.float32),
            pltpu.VMEM((_CHUNK,), jnp.float32),
            pltpu.VMEM((_CHUNK,), jnp.float32),
            pltpu.VMEM((_NB * 1024,), jnp.int32),
            pltpu.VMEM((_NB * 1024,), jnp.float32),
            pltpu.VMEM((_NB * 1024,), jnp.float32),
            pltpu.VMEM((_CHUNK,), jnp.float32),
            pltpu.VMEM((_CHUNK,), jnp.float32),
            pltpu.VMEM((_CHUNK,), jnp.float32),
            pltpu.VMEM((_NB * 1024,), jnp.int32),
            pltpu.VMEM((_NB * 1024,), jnp.float32),
            pltpu.VMEM((_NB * 1024,), jnp.float32),
            pltpu.SemaphoreType.DMA,
            pltpu.SemaphoreType.DMA,
            pltpu.SemaphoreType.DMA,
            pltpu.SemaphoreType.DMA,
            pltpu.SemaphoreType.DMA,
            pltpu.SemaphoreType.DMA,
        ],
    )(_sc_body)
    tabflat = (table.reshape(16384, 128, 2, 8)
               .transpose(2, 0, 3, 1)
               .reshape(2, _PLANE_WORDS))
    out2 = run(xyz[:, 0], xyz[:, 1], xyz[:, 2], tabflat)
    out4 = out2.reshape(2, _N // 128, 8, 128)
    return out4.transpose(1, 3, 0, 2).reshape(_N, _D)
